# baseline (speedup 1.0000x reference)
import jax
import jax.numpy as jnp
from jax import lax
from jax.experimental import pallas as pl
from jax.experimental.pallas import tpu as pltpu
from jax.experimental.pallas import tpu_sc as plsc

NC = 2
NS = 16
NW = NC * NS

NB = 4096
NT = 200
D = 64
BK = NB // NW
LG = BK // 16


def _body(widx_hbm, pidx_hbm, word_hbm, pos_hbm, out_hbm,
          widx_v, pidx_v, pos_stage, pos_sh, wbuf, tbuf,
          sem_g, sem_p, sem_w):
    b0 = (lax.axis_index("s") * NC + lax.axis_index("c")) * BK
    wid = b0 // BK
    iota = lax.iota(jnp.int32, 16)

    pltpu.sync_copy(widx_hbm.at[:, pl.ds(b0, BK)], widx_v)
    pltpu.sync_copy(pidx_hbm.at[:, pl.ds(b0, BK)], pidx_v)

    @pl.when(lax.axis_index("s") == 0)
    def _():
        pltpu.sync_copy(pos_hbm, pos_stage)
        pltpu.sync_copy(pos_stage, pos_sh)

    plsc.subcore_barrier()

    def fire_word(t, p):
        pltpu.async_copy(word_hbm.at[widx_v.at[t]], wbuf.at[p], sem_g.at[p])

    def wait_word(p):
        pltpu.make_async_copy(word_hbm.at[widx_v.at[0]], wbuf.at[p],
                              sem_g.at[p]).wait()

    def fire_pos(t, p):
        pltpu.async_copy(pos_sh.at[pidx_v.at[t]], wbuf.at[p], sem_p.at[p],
                         add=True)

    def wait_pos(p):
        pltpu.make_async_copy(pos_sh.at[pidx_v.at[0]], wbuf.at[p],
                              sem_p.at[p]).wait()

    def fire_write(t, p):
        for dt in range(8):
            pltpu.async_copy(tbuf.at[p, pl.ds(dt * 8, 8)],
                             out_hbm.at[t, dt, wid], sem_w.at[p])

    def wait_write(p):
        for dt in range(8):
            pltpu.make_async_copy(tbuf.at[p, pl.ds(dt * 8, 8)],
                                  out_hbm.at[0, dt, 0], sem_w.at[p]).wait()

    def compute(t, p, lgs):
        for lg in lgs:
            b16 = iota + (lg * 16)

            def k_body(k, carry):
                ix = jnp.full((16,), k, jnp.int32) ^ iota
                dvs, wvs = [], []
                for db in range(0, D, 16):
                    dv = ix + db
                    dvs.append(dv)
                    wvs.append(plsc.load_gather(wbuf.at[p], [b16, dv]))
                for j in range(4):
                    plsc.store_scatter(tbuf.at[p], [dvs[j], b16], wvs[j])
                return carry

            lax.fori_loop(0, 16, k_body, 0, unroll=2)

    fire_word(0, 0)
    wait_word(0)
    fire_pos(0, 0)

    def step(t, p):
        q = (p + 1) % 2

        @pl.when(t + 1 < NT)
        def _():
            fire_word(t + 1, q)

        @pl.when(jnp.logical_and(t >= 2, t < NT + 2))
        def _():
            wait_write(p)

        @pl.when(t < NT)
        def _():
            wait_pos(p)

        @pl.when(t < NT)
        def _():
            compute(t, p, range(0, LG // 2))

        @pl.when(t + 1 < NT)
        def _():
            wait_word(q)
            fire_pos(t + 1, q)

        @pl.when(t < NT)
        def _():
            compute(t, p, range(LG // 2, LG))
            fire_write(t, p)

    def outer(g, carry):
        step(g * 2, 0)
        step(g * 2 + 1, 1)
        return carry

    lax.fori_loop(0, (NT + 2) // 2, outer, 0)


@jax.jit
def _emb(widx2, pidx2, word_table, pos_table):
    mesh = plsc.VectorSubcoreMesh(
        core_axis_name="c", subcore_axis_name="s",
        num_cores=NC, num_subcores=NS)
    f = pl.kernel(
        _body,
        out_type=jax.ShapeDtypeStruct((NT, D // 8, NW, 8, BK), jnp.float32),
        mesh=mesh,
        compiler_params=pltpu.CompilerParams(
            use_tc_tiling_on_sc=False, needs_layout_passes=False),
        scratch_types=[
            pltpu.VMEM((NT, BK), jnp.int32),
            pltpu.VMEM((NT, BK), jnp.int32),
            pltpu.VMEM((NT, D), jnp.float32),
            pltpu.VMEM_SHARED((NT, D), jnp.float32),
            pltpu.VMEM((2, BK, D), jnp.float32),
            pltpu.VMEM((2, D, BK), jnp.float32),
            pltpu.SemaphoreType.DMA((2,)),
            pltpu.SemaphoreType.DMA((2,)),
            pltpu.SemaphoreType.DMA((2,)),
        ],
    )
    return f(widx2, pidx2, word_table, pos_table)


def kernel(input_ids, pos_ids, word_table, pos_table):
    widx2 = input_ids.T.astype(jnp.int32)
    pidx2 = pos_ids.T.astype(jnp.int32)
    x5 = _emb(widx2, pidx2, word_table, pos_table)
    return x5.transpose(2, 4, 0, 1, 3).reshape(NB, NT, D)


# speedup vs baseline: 1.1449x; 1.1449x over previous
import jax
import jax.numpy as jnp
from jax import lax
from jax.experimental import pallas as pl
from jax.experimental.pallas import tpu as pltpu
from jax.experimental.pallas import tpu_sc as plsc

NC = 2
NS = 16
NW = NC * NS

NB = 4096
NT = 200
D = 64
BK = NB // NW
LG = BK // 16


def _body(widx_hbm, pidx_hbm, word_hbm, pos_hbm, out_hbm,
          widx_v, pidx_v, pos_stage, pos_sh, wbuf, tbuf,
          sem_g, sem_p, sem_w):
    b0 = (lax.axis_index("s") * NC + lax.axis_index("c")) * BK
    wid = b0 // BK
    iota = lax.iota(jnp.int32, 16)

    pltpu.sync_copy(widx_hbm.at[:, pl.ds(b0, BK)], widx_v)
    pltpu.sync_copy(pidx_hbm.at[:, pl.ds(b0, BK)], pidx_v)

    @pl.when(lax.axis_index("s") == 0)
    def _():
        pltpu.sync_copy(pos_hbm, pos_stage)
        pltpu.sync_copy(pos_stage, pos_sh)

    plsc.subcore_barrier()

    def fire_word(t, p):
        pltpu.async_copy(word_hbm.at[widx_v.at[t]], wbuf.at[p], sem_g.at[p])

    def wait_word(p):
        pltpu.make_async_copy(word_hbm.at[widx_v.at[0]], wbuf.at[p],
                              sem_g.at[p]).wait()

    def fire_pos(t, p):
        pltpu.async_copy(pos_sh.at[pidx_v.at[t]], wbuf.at[p], sem_p.at[p],
                         add=True)

    def wait_pos(p):
        pltpu.make_async_copy(pos_sh.at[pidx_v.at[0]], wbuf.at[p],
                              sem_p.at[p]).wait()

    def fire_write(t, p):
        for dt in range(8):
            pltpu.async_copy(tbuf.at[p, pl.ds(dt * 8, 8)],
                             out_hbm.at[t, dt, wid], sem_w.at[p])

    def wait_write(p):
        for dt in range(8):
            pltpu.make_async_copy(tbuf.at[p, pl.ds(dt * 8, 8)],
                                  out_hbm.at[0, dt, 0], sem_w.at[p]).wait()

    def compute(t, p):
        for lg in range(LG):
            b16 = iota + (lg * 16)

            def k_body(k, carry):
                ix = jnp.full((16,), k, jnp.int32) ^ iota
                dvs, wvs = [], []
                for db in range(0, D, 16):
                    dv = ix + db
                    dvs.append(dv)
                    wvs.append(plsc.load_gather(wbuf.at[p], [b16, dv]))
                for j in range(4):
                    plsc.store_scatter(tbuf.at[p], [dvs[j], b16], wvs[j])
                return carry

            lax.fori_loop(0, 16, k_body, 0, unroll=2)

    fire_word(0, 0)
    wait_word(0)
    fire_pos(0, 0)

    def step(t, p):
        q = (p + 1) % 2

        @pl.when(t + 1 < NT)
        def _():
            fire_word(t + 1, q)

        @pl.when(jnp.logical_and(t >= 2, t < NT + 2))
        def _():
            wait_write(p)

        @pl.when(t < NT)
        def _():
            wait_pos(p)

        @pl.when(t < NT)
        def _():
            compute(t, p)
            fire_write(t, p)

        @pl.when(t + 1 < NT)
        def _():
            wait_word(q)
            fire_pos(t + 1, q)

    def outer(g, carry):
        step(g * 2, 0)
        step(g * 2 + 1, 1)
        return carry

    lax.fori_loop(0, (NT + 2) // 2, outer, 0)


@jax.jit
def _emb(widx2, pidx2, word_table, pos_table):
    mesh = plsc.VectorSubcoreMesh(
        core_axis_name="c", subcore_axis_name="s",
        num_cores=NC, num_subcores=NS)
    f = pl.kernel(
        _body,
        out_type=jax.ShapeDtypeStruct((NT, D // 8, NW, 8, BK), jnp.float32),
        mesh=mesh,
        compiler_params=pltpu.CompilerParams(
            use_tc_tiling_on_sc=False, needs_layout_passes=False),
        scratch_types=[
            pltpu.VMEM((NT, BK), jnp.int32),
            pltpu.VMEM((NT, BK), jnp.int32),
            pltpu.VMEM((NT, D), jnp.float32),
            pltpu.VMEM_SHARED((NT, D), jnp.float32),
            pltpu.VMEM((2, BK, D), jnp.float32),
            pltpu.VMEM((2, D, BK), jnp.float32),
            pltpu.SemaphoreType.DMA((2,)),
            pltpu.SemaphoreType.DMA((2,)),
            pltpu.SemaphoreType.DMA((2,)),
        ],
    )
    return f(widx2, pidx2, word_table, pos_table)


def kernel(input_ids, pos_ids, word_table, pos_table):
    widx2 = input_ids.T.astype(jnp.int32)
    pidx2 = pos_ids.T.astype(jnp.int32)
    x5 = _emb(widx2, pidx2, word_table, pos_table)
    return x5.transpose(2, 4, 0, 1, 3).reshape(NB, NT, D)
